# baseline (speedup 1.0000x reference)
import functools
import math

import jax
import jax.numpy as jnp
import numpy as np
from jax import lax
from jax.experimental import pallas as pl
from jax.experimental.pallas import tpu as pltpu
from jax.experimental.pallas import tpu_sc as plsc

_MAX_LEN = 200
_HID = 64
_BATCH = 4096
_VOCAB = 1000000
_SCALE = math.sqrt(_HID)

_NW = 32
_BB = _BATCH // _NW
_LANES = 16
_NJ = _BB // _LANES

_NFULL = _VOCAB // 128
_TAILC = _VOCAB - _NFULL * 128
_KMAX = 245
_RA = 4
_RB = 3


def _pos_flat_np():
    para = np.arange(_MAX_LEN, dtype=np.float32).reshape(-1, 1) / np.power(
        10000.0, np.arange(0, _HID, 2, dtype=np.float32) / _HID)
    pos = np.zeros((_MAX_LEN, _HID), dtype=np.float32)
    pos[:, 0::2] = np.sin(para)
    pos[:, 1::2] = np.cos(para)
    return pos.reshape(-1)


_POS_FLAT = _pos_flat_np()


def _conv_body(tt_hbm, out_hbm, src_v, dst_v, tl_v, *sems):
    wid = lax.axis_index("s") * 2 + lax.axis_index("c")
    sr = sems[:_RA]
    sw = sems[_RA:2 * _RA]
    iota = lax.broadcasted_iota(jnp.int32, (_LANES,), 0)
    row_const = [iota + (16 * k8) % 64 for k8 in range(8)]
    col_const = [iota + 16 * k8 for k8 in range(8)]

    def wof(k):
        return wid + 32 * k

    def fire_read(k, b):
        pltpu.async_copy(tt_hbm.at[:, pl.ds(128 * wof(k), 128)],
                         src_v.at[b], sr[b])

    def drain_read(k, b):
        pltpu.make_async_copy(tt_hbm.at[:, pl.ds(128 * wof(k), 128)],
                              src_v.at[b], sr[b]).wait()

    def drain_write(k, b):
        pltpu.make_async_copy(dst_v.at[b],
                              out_hbm.at[pl.ds(64 * wof(k), 64), :],
                              sw[b]).wait()

    for k0 in range(_RA - 1):
        fire_read(k0, k0)

    def ring_body(pp, carry):
        k0 = pp * _RA
        for b in range(_RA):
            k = k0 + b
            w = wof(k)

            @pl.when(w < _NFULL)
            def _body():
                @pl.when(wof(k + _RA - 1) < _NFULL)
                def _():
                    fire_read(k + _RA - 1, (b + _RA - 1) % _RA)
                @pl.when(k >= _RA)
                def _():
                    drain_write(k - _RA, b)

                drain_read(k, b)
                src2d = src_v.at[b]
                dst2d = dst_v.at[b]

                def jj_body(jj0, jc):
                    jjv = jnp.bitwise_and(jj0 + iota, 63)
                    col0 = jjv + jjv
                    col1 = col0 + 1
                    for k8 in range(8):
                        col = col0 if k8 < 4 else col1
                        val = plsc.load_gather(src2d, [row_const[k8], col])
                        plsc.store_scatter(dst2d, [jjv, col_const[k8]], val)
                    return jc

                lax.fori_loop(0, 64, jj_body, 0, unroll=4)
                pltpu.async_copy(dst_v.at[b],
                                 out_hbm.at[pl.ds(64 * w, 64), :], sw[b])
        return carry

    lax.fori_loop(0, (_KMAX + _RA - 1) // _RA + 1, ring_body, 0)
    for ke in range(240, 245):
        w_e = wof(ke)

        @pl.when((w_e < _NFULL) & (w_e + 32 * _RA >= _NFULL))
        def _():
            drain_write(ke, ke % _RA)

    @pl.when(wid == (_NFULL % _NW))
    def _tail():
        pltpu.sync_copy(tt_hbm.at[:, pl.ds(128 * _NFULL, _TAILC)], tl_v)

        def jj_body(jj, jc):
            c0 = jnp.full((_LANES,), 2 * jj, jnp.int32)
            c1 = jnp.full((_LANES,), 2 * jj + 1, jnp.int32)
            for k8 in range(8):
                col = c0 if k8 < 4 else c1
                val = plsc.load_gather(tl_v, [row_const[k8], col])
                dst_v[0, jj, pl.ds(16 * k8, _LANES)] = val
            return jc

        lax.fori_loop(0, _TAILC // 2, jj_body, 0)
        pltpu.sync_copy(dst_v.at[0].at[pl.ds(0, _TAILC // 2), :],
                        out_hbm.at[pl.ds(64 * _NFULL, _TAILC // 2), :])


def _sc_body(tbl_hbm, xt_hbm, pos_hbm, out_hbm,
             xs_v, idxp_v, offs_v, rows_v, ob_v, pos_v, *sems):
    wid = lax.axis_index("s") * 2 + lax.axis_index("c")
    b0 = wid * _BB
    sg = sems[:_RB]
    so = sems[_RB:2 * _RB]
    pltpu.sync_copy(pos_hbm, pos_v)
    pltpu.sync_copy(xt_hbm.at[:, pl.ds(b0, _BB)], xs_v)
    iota = lax.broadcasted_iota(jnp.int32, (_LANES,), 0)

    def stage_and_fire(p, nb):
        for j in range(_NJ):
            sl = pl.ds(j * _LANES, _LANES)
            v = xs_v[p, sl]
            idxp_v[nb, sl] = lax.shift_right_logical(v, 1)
            offs_v[nb, sl] = lax.shift_left(jnp.bitwise_and(v, 1), 6)
        pltpu.async_copy(tbl_hbm.at[idxp_v.at[nb]], rows_v.at[nb], sg[nb])

    def drain_gather(b):
        pltpu.make_async_copy(tbl_hbm.at[idxp_v.at[b]], rows_v.at[b],
                              sg[b]).wait()

    def drain_out(p, b):
        pltpu.make_async_copy(ob_v.at[b],
                              out_hbm.at[p, :, pl.ds(b0, _BB)], so[b]).wait()

    for p0 in range(_RB - 1):
        stage_and_fire(p0, p0)

    def ring_body(pp, carry):
        p0 = pp * _RB
        for b in range(_RB):
            p = p0 + b

            @pl.when(p < _MAX_LEN)
            def _body():
                @pl.when(p + _RB - 1 < _MAX_LEN)
                def _():
                    @pl.when(p >= 1)
                    def _():
                        drain_out(p - 1, (b - 1) % _RB)
                    stage_and_fire(p + _RB - 1, (b + _RB - 1) % _RB)

                drain_gather(b)

                rows2d = rows_v.at[b]
                ob2d = ob_v.at[b]
                offs = [offs_v[b, pl.ds(j * _LANES, _LANES)]
                        for j in range(_NJ)]
                rowid = [j * _LANES + iota for j in range(_NJ)]
                pbase = p * _HID

                def h_body(h0, hc):
                    hv = jnp.bitwise_and(h0 + iota, _HID - 1)
                    pv = plsc.load_gather(pos_v, [pbase + hv])
                    for j in range(_NJ):
                        col = offs[j] + hv
                        val = plsc.load_gather(rows2d, [rowid[j], col])
                        plsc.store_scatter(ob2d, [hv, rowid[j]],
                                           val * _SCALE + pv)
                    return hc

                lax.fori_loop(0, _HID, h_body, 0, unroll=4)
                pltpu.async_copy(ob_v.at[b],
                                 out_hbm.at[p, :, pl.ds(b0, _BB)], so[b])
        return carry

    lax.fori_loop(0, _MAX_LEN // _RB + 1, ring_body, 0)
    for pe in (_MAX_LEN - 3, _MAX_LEN - 2, _MAX_LEN - 1):
        drain_out(pe, pe % _RB)


@jax.jit
def _run(tt, xt, pos):
    mesh = plsc.VectorSubcoreMesh(core_axis_name="c", subcore_axis_name="s")
    cp = pltpu.CompilerParams(use_tc_tiling_on_sc=True,
                              needs_layout_passes=False)
    conv = functools.partial(
        pl.kernel, mesh=mesh,
        out_type=jax.ShapeDtypeStruct((_VOCAB // 2, 2 * _HID), jnp.float32),
        scratch_types=(
            [pltpu.VMEM((_RA, _HID, 128), jnp.float32),
             pltpu.VMEM((_RA, _HID, 128), jnp.float32),
             pltpu.VMEM((_HID, _TAILC), jnp.float32)]
            + [pltpu.SemaphoreType.DMA] * (2 * _RA)
        ),
        compiler_params=cp,
    )(_conv_body)
    tbl2 = conv(tt)
    gath = functools.partial(
        pl.kernel, mesh=mesh,
        out_type=jax.ShapeDtypeStruct((_MAX_LEN, _HID, _BATCH), jnp.float32),
        scratch_types=(
            [pltpu.VMEM((_MAX_LEN, _BB), jnp.int32),
             pltpu.VMEM((_RB, _BB), jnp.int32),
             pltpu.VMEM((_RB, _BB), jnp.int32),
             pltpu.VMEM((_RB, _BB, 2 * _HID), jnp.float32),
             pltpu.VMEM((_RB, _HID, _BB), jnp.float32),
             pltpu.VMEM((_MAX_LEN * _HID,), jnp.float32)]
            + [pltpu.SemaphoreType.DMA] * (2 * _RB)
        ),
        compiler_params=cp,
    )(_sc_body)
    return gath(tbl2, xt, pos)


def kernel(x, table):
    xt = x.T.astype(jnp.int32)
    tt = table.T
    pos = jnp.asarray(_POS_FLAT)
    o = _run(tt, xt, pos)
    return jnp.transpose(o, (2, 0, 1))


# speedup vs baseline: 2.2794x; 2.2794x over previous
import functools
import math

import jax
import jax.numpy as jnp
import numpy as np
from jax import lax
from jax.experimental import pallas as pl
from jax.experimental.pallas import tpu as pltpu
from jax.experimental.pallas import tpu_sc as plsc

_MAX_LEN = 200
_HID = 64
_BATCH = 4096
_VOCAB = 1000000
_SCALE = math.sqrt(_HID)

_NW = 32
_BB = _BATCH // _NW
_LANES = 16
_NJ = _BB // _LANES

_NFULL = _VOCAB // 128
_TAILC = _VOCAB - _NFULL * 128
_KMAX = 245
_RA = 4
_RB = 3


def _pos_flat_np():
    para = np.arange(_MAX_LEN, dtype=np.float32).reshape(-1, 1) / np.power(
        10000.0, np.arange(0, _HID, 2, dtype=np.float32) / _HID)
    pos = np.zeros((_MAX_LEN, _HID), dtype=np.float32)
    pos[:, 0::2] = np.sin(para)
    pos[:, 1::2] = np.cos(para)
    return pos.reshape(-1)


_POS_FLAT = _pos_flat_np()


def _conv_body(tt_hbm, out_hbm, src_v, dst_v, tl_v, *sems):
    wid = lax.axis_index("s") * 2 + lax.axis_index("c")
    sr = sems[:_RA]
    sw = sems[_RA:2 * _RA]
    iota = lax.broadcasted_iota(jnp.int32, (_LANES,), 0)
    row_const = [iota + (16 * k8) % 64 for k8 in range(8)]
    col_const = [iota + 16 * k8 for k8 in range(8)]

    def wof(k):
        return wid + 32 * k

    def fire_read(k, b):
        pltpu.async_copy(tt_hbm.at[:, pl.ds(128 * wof(k), 128)],
                         src_v.at[b], sr[b])

    def drain_read(k, b):
        pltpu.make_async_copy(tt_hbm.at[:, pl.ds(128 * wof(k), 128)],
                              src_v.at[b], sr[b]).wait()

    def drain_write(k, b):
        pltpu.make_async_copy(dst_v.at[b],
                              out_hbm.at[pl.ds(64 * wof(k), 64), :],
                              sw[b]).wait()

    for k0 in range(_RA - 1):
        fire_read(k0, k0)

    def ring_body(pp, carry):
        k0 = pp * _RA
        for b in range(_RA):
            k = k0 + b
            w = wof(k)

            @pl.when((k >= 1) & (wof(k - 1) < _NFULL))
            def _fire_prev():
                pltpu.async_copy(dst_v.at[(b - 1) % _RA],
                                 out_hbm.at[pl.ds(64 * wof(k - 1), 64), :],
                                 sw[(b - 1) % _RA])

            @pl.when(w < _NFULL)
            def _body():
                @pl.when(wof(k + _RA - 1) < _NFULL)
                def _():
                    fire_read(k + _RA - 1, (b + _RA - 1) % _RA)
                @pl.when(k >= _RA)
                def _():
                    drain_write(k - _RA, b)

                drain_read(k, b)
                src2d = src_v.at[b]
                dst2d = dst_v.at[b]

                @plsc.parallel_loop(0, 64, 1, unroll=4)
                def jj_body(jj0):
                    jjv = jnp.bitwise_and(jj0 + iota, 63)
                    col0 = jjv + jjv
                    col1 = col0 + 1
                    for k8 in range(8):
                        col = col0 if k8 < 4 else col1
                        val = plsc.load_gather(src2d, [row_const[k8], col])
                        plsc.store_scatter(dst2d, [jjv, col_const[k8]], val)
        return carry

    lax.fori_loop(0, (_KMAX + _RA - 1) // _RA + 1, ring_body, 0)
    for ke in range(240, 245):
        w_e = wof(ke)

        @pl.when((w_e < _NFULL) & (w_e + 32 * _RA >= _NFULL))
        def _():
            drain_write(ke, ke % _RA)

    @pl.when(wid == (_NFULL % _NW))
    def _tail():
        pltpu.sync_copy(tt_hbm.at[:, pl.ds(128 * _NFULL, _TAILC)], tl_v)

        def jj_body(jj, jc):
            c0 = jnp.full((_LANES,), 2 * jj, jnp.int32)
            c1 = jnp.full((_LANES,), 2 * jj + 1, jnp.int32)
            for k8 in range(8):
                col = c0 if k8 < 4 else c1
                val = plsc.load_gather(tl_v, [row_const[k8], col])
                dst_v[0, jj, pl.ds(16 * k8, _LANES)] = val
            return jc

        lax.fori_loop(0, _TAILC // 2, jj_body, 0)
        pltpu.sync_copy(dst_v.at[0].at[pl.ds(0, _TAILC // 2), :],
                        out_hbm.at[pl.ds(64 * _NFULL, _TAILC // 2), :])


def _sc_body(tbl_hbm, xt_hbm, pos_hbm, out_hbm,
             xs_v, idxp_v, offs_v, rows_v, ob_v, pos_v, *sems):
    wid = lax.axis_index("s") * 2 + lax.axis_index("c")
    b0 = wid * _BB
    sg = sems[:_RB]
    so = sems[_RB:2 * _RB]
    pltpu.sync_copy(pos_hbm, pos_v)
    pltpu.sync_copy(xt_hbm.at[:, pl.ds(b0, _BB)], xs_v)
    iota = lax.broadcasted_iota(jnp.int32, (_LANES,), 0)

    def stage_and_fire(p, nb):
        for j in range(_NJ):
            sl = pl.ds(j * _LANES, _LANES)
            v = xs_v[p, sl]
            idxp_v[nb, sl] = lax.shift_right_logical(v, 1)
            offs_v[nb, sl] = lax.shift_left(jnp.bitwise_and(v, 1), 6)
        pltpu.async_copy(tbl_hbm.at[idxp_v.at[nb]], rows_v.at[nb], sg[nb])

    def drain_gather(b):
        pltpu.make_async_copy(tbl_hbm.at[idxp_v.at[b]], rows_v.at[b],
                              sg[b]).wait()

    def drain_out(p, b):
        pltpu.make_async_copy(ob_v.at[b],
                              out_hbm.at[p, :, pl.ds(b0, _BB)], so[b]).wait()

    for p0 in range(_RB - 1):
        stage_and_fire(p0, p0)

    def ring_body(pp, carry):
        p0 = pp * _RB
        for b in range(_RB):
            p = p0 + b

            @pl.when((p >= 1) & (p - 1 < _MAX_LEN))
            def _fire_prev():
                pltpu.async_copy(ob_v.at[(b - 1) % _RB],
                                 out_hbm.at[p - 1, :, pl.ds(b0, _BB)],
                                 so[(b - 1) % _RB])

            @pl.when((p >= 2) & (p < _MAX_LEN))
            def _drain_prev():
                drain_out(p - 2, (b - 2) % _RB)

            @pl.when(p < _MAX_LEN)
            def _body():
                @pl.when(p + _RB - 1 < _MAX_LEN)
                def _():
                    stage_and_fire(p + _RB - 1, (b + _RB - 1) % _RB)

                drain_gather(b)

                rows2d = rows_v.at[b]
                ob2d = ob_v.at[b]
                offs = [offs_v[b, pl.ds(j * _LANES, _LANES)]
                        for j in range(_NJ)]
                rowid = [j * _LANES + iota for j in range(_NJ)]
                pbase = p * _HID

                @plsc.parallel_loop(0, _HID, 1, unroll=4)
                def h_body(h0):
                    hv = jnp.bitwise_and(h0 + iota, _HID - 1)
                    pv = plsc.load_gather(pos_v, [pbase + hv])
                    for j in range(_NJ):
                        col = offs[j] + hv
                        val = plsc.load_gather(rows2d, [rowid[j], col])
                        plsc.store_scatter(ob2d, [hv, rowid[j]],
                                           val * _SCALE + pv)
        return carry

    lax.fori_loop(0, _MAX_LEN // _RB + 1, ring_body, 0)
    for pe in (_MAX_LEN - 2, _MAX_LEN - 1):
        drain_out(pe, pe % _RB)


@jax.jit
def _run(tt, xt, pos):
    mesh = plsc.VectorSubcoreMesh(core_axis_name="c", subcore_axis_name="s")
    cp = pltpu.CompilerParams(use_tc_tiling_on_sc=True,
                              needs_layout_passes=False)
    conv = functools.partial(
        pl.kernel, mesh=mesh,
        out_type=jax.ShapeDtypeStruct((_VOCAB // 2, 2 * _HID), jnp.float32),
        scratch_types=(
            [pltpu.VMEM((_RA, _HID, 128), jnp.float32),
             pltpu.VMEM((_RA, _HID, 128), jnp.float32),
             pltpu.VMEM((_HID, _TAILC), jnp.float32)]
            + [pltpu.SemaphoreType.DMA] * (2 * _RA)
        ),
        compiler_params=cp,
    )(_conv_body)
    tbl2 = conv(tt)
    gath = functools.partial(
        pl.kernel, mesh=mesh,
        out_type=jax.ShapeDtypeStruct((_MAX_LEN, _HID, _BATCH), jnp.float32),
        scratch_types=(
            [pltpu.VMEM((_MAX_LEN, _BB), jnp.int32),
             pltpu.VMEM((_RB, _BB), jnp.int32),
             pltpu.VMEM((_RB, _BB), jnp.int32),
             pltpu.VMEM((_RB, _BB, 2 * _HID), jnp.float32),
             pltpu.VMEM((_RB, _HID, _BB), jnp.float32),
             pltpu.VMEM((_MAX_LEN * _HID,), jnp.float32)]
            + [pltpu.SemaphoreType.DMA] * (2 * _RB)
        ),
        compiler_params=cp,
    )(_sc_body)
    return gath(tbl2, xt, pos)


def kernel(x, table):
    xt = x.T.astype(jnp.int32)
    tt = table.T
    pos = jnp.asarray(_POS_FLAT)
    o = _run(tt, xt, pos)
    return jnp.transpose(o, (2, 0, 1))
